# baseline (speedup 1.0000x reference)
import jax
import jax.numpy as jnp
from jax.experimental import pallas as pl
from jax.experimental.pallas import tpu as pltpu

_LEVELS = 256
_DIST = 5
_ROWS = 8


def _hist_kernel(w, img_ref, cnt_ref):
    c = pl.program_id(1)

    @pl.when(c == 0)
    def _():
        cnt_ref[...] = jnp.zeros_like(cnt_ref)

    p = _ROWS * w
    blk = img_ref[0]
    q = jnp.clip(jnp.floor(blk * 255.0), 0.0, 255.0)
    gray = 0.299 * q[0:1] + 0.587 * q[1:2] + 0.114 * q[2:3]
    gidx = jnp.clip(jnp.round(gray), 0.0, 255.0).astype(jnp.int32)

    pos = jax.lax.broadcasted_iota(jnp.int32, (1, p), 1)
    col = jax.lax.rem(pos, w)
    i_val = jnp.where(col < w - _DIST, gidx, -1)
    j_val = jnp.concatenate(
        [gidx[:, _DIST:], jnp.zeros((1, _DIST), jnp.int32)], axis=1)

    lvl = jax.lax.broadcasted_iota(jnp.int32, (_LEVELS, p), 0)
    oh_i = (i_val == lvl).astype(jnp.bfloat16)
    oh_j = (j_val == lvl).astype(jnp.bfloat16)

    cnt_ref[0] += jax.lax.dot_general(
        oh_i, oh_j, (((1,), (1,)), ((), ())),
        preferred_element_type=jnp.float32)


def _stats_kernel(cnt_ref, out_ref):
    cnt = cnt_ref[0]
    glcm = cnt + cnt.T
    glcm = glcm / jnp.sum(glcm)

    lv = jax.lax.broadcasted_iota(
        jnp.int32, (_LEVELS, _LEVELS), 0).astype(jnp.float32)
    lh = jax.lax.broadcasted_iota(
        jnp.int32, (_LEVELS, _LEVELS), 1).astype(jnp.float32)
    di = lv - lh

    contrast = jnp.sum(glcm * (di * di))
    dissimilarity = jnp.sum(glcm * jnp.abs(di))
    homogeneity = jnp.sum(glcm / (1.0 + di * di))
    energy = jnp.sqrt(jnp.sum(glcm * glcm))

    mu_i = jnp.sum(glcm * lv)
    mu_j = jnp.sum(glcm * lh)
    ci = lv - mu_i
    cj = lh - mu_j
    var_i = jnp.sum(glcm * ci * ci)
    var_j = jnp.sum(glcm * cj * cj)
    cov = jnp.sum(glcm * ci * cj)
    std = jnp.sqrt(var_i * var_j)
    corr = jnp.where(std < 1e-15, 1.0, cov / jnp.maximum(std, 1e-15))

    fmin = jnp.minimum(
        jnp.minimum(jnp.minimum(contrast, dissimilarity),
                    jnp.minimum(homogeneity, energy)), corr)
    fmax = jnp.maximum(
        jnp.maximum(jnp.maximum(contrast, dissimilarity),
                    jnp.maximum(homogeneity, energy)), corr)
    lane = jax.lax.broadcasted_iota(jnp.int32, (1, 128), 1)
    feat = jnp.where(lane == 0, contrast,
           jnp.where(lane == 1, dissimilarity,
           jnp.where(lane == 2, homogeneity,
           jnp.where(lane == 3, energy, corr))))
    fn = (feat - fmin) / (fmax - fmin)
    out_ref[0] = jnp.floor(fn * 255.0) / 255.0


@jax.jit
def kernel(img):
    b, _, h, w = img.shape
    nchunks = h // _ROWS
    flat = img.reshape(b, 3, h * w)

    counts = pl.pallas_call(
        lambda img_ref, cnt_ref: _hist_kernel(w, img_ref, cnt_ref),
        grid=(b, nchunks),
        in_specs=[pl.BlockSpec((1, 3, _ROWS * w), lambda i, c: (i, 0, c))],
        out_specs=pl.BlockSpec((1, _LEVELS, _LEVELS), lambda i, c: (i, 0, 0)),
        out_shape=jax.ShapeDtypeStruct((b, _LEVELS, _LEVELS), jnp.float32),
        compiler_params=pltpu.CompilerParams(
            dimension_semantics=("parallel", "arbitrary")),
    )(flat)

    out = pl.pallas_call(
        _stats_kernel,
        grid=(b,),
        in_specs=[pl.BlockSpec((1, _LEVELS, _LEVELS), lambda i: (i, 0, 0))],
        out_specs=pl.BlockSpec((1, 1, 128), lambda i: (i, 0, 0)),
        out_shape=jax.ShapeDtypeStruct((b, 1, 128), jnp.float32),
        compiler_params=pltpu.CompilerParams(
            dimension_semantics=("parallel",)),
    )(counts)

    feats = out[:, 0, :5]
    return jnp.broadcast_to(feats[:, None, :], (b, 3, 5))


# speedup vs baseline: 1.3408x; 1.3408x over previous
import jax
import jax.numpy as jnp
from jax.experimental import pallas as pl
from jax.experimental.pallas import tpu as pltpu

_LEVELS = 256
_DIST = 5
_ROWS = 32


def _hist_kernel(w, img_ref, cnt_ref):
    c = pl.program_id(1)

    @pl.when(c == 0)
    def _():
        cnt_ref[...] = jnp.zeros_like(cnt_ref)

    p = _ROWS * w
    blk = img_ref[0]
    q = jnp.clip(jnp.floor(blk * 255.0), 0.0, 255.0)
    gray = 0.299 * q[0:1] + 0.587 * q[1:2] + 0.114 * q[2:3]
    gidx = jnp.clip(jnp.round(gray), 0.0, 255.0).astype(jnp.int32)

    pos = jax.lax.broadcasted_iota(jnp.int32, (1, p), 1)
    col = jax.lax.rem(pos, w)
    i_val = jnp.where(col < w - _DIST, gidx, -1)
    j_val = jnp.concatenate(
        [gidx[:, _DIST:], jnp.zeros((1, _DIST), jnp.int32)], axis=1)

    lvl = jax.lax.broadcasted_iota(jnp.int32, (_LEVELS, p), 0)
    oh_i = (i_val == lvl).astype(jnp.bfloat16)
    oh_j = (j_val == lvl).astype(jnp.bfloat16)

    cnt_ref[0] += jax.lax.dot_general(
        oh_i, oh_j, (((1,), (1,)), ((), ())),
        preferred_element_type=jnp.float32)


def _stats_kernel(cnt_ref, out_ref):
    cnt = cnt_ref[0]
    glcm = cnt + cnt.T
    glcm = glcm / jnp.sum(glcm)

    lv = jax.lax.broadcasted_iota(
        jnp.int32, (_LEVELS, _LEVELS), 0).astype(jnp.float32)
    lh = jax.lax.broadcasted_iota(
        jnp.int32, (_LEVELS, _LEVELS), 1).astype(jnp.float32)
    di = lv - lh

    contrast = jnp.sum(glcm * (di * di))
    dissimilarity = jnp.sum(glcm * jnp.abs(di))
    homogeneity = jnp.sum(glcm / (1.0 + di * di))
    energy = jnp.sqrt(jnp.sum(glcm * glcm))

    mu_i = jnp.sum(glcm * lv)
    mu_j = jnp.sum(glcm * lh)
    ci = lv - mu_i
    cj = lh - mu_j
    var_i = jnp.sum(glcm * ci * ci)
    var_j = jnp.sum(glcm * cj * cj)
    cov = jnp.sum(glcm * ci * cj)
    std = jnp.sqrt(var_i * var_j)
    corr = jnp.where(std < 1e-15, 1.0, cov / jnp.maximum(std, 1e-15))

    fmin = jnp.minimum(
        jnp.minimum(jnp.minimum(contrast, dissimilarity),
                    jnp.minimum(homogeneity, energy)), corr)
    fmax = jnp.maximum(
        jnp.maximum(jnp.maximum(contrast, dissimilarity),
                    jnp.maximum(homogeneity, energy)), corr)
    lane = jax.lax.broadcasted_iota(jnp.int32, (1, 128), 1)
    feat = jnp.where(lane == 0, contrast,
           jnp.where(lane == 1, dissimilarity,
           jnp.where(lane == 2, homogeneity,
           jnp.where(lane == 3, energy, corr))))
    fn = (feat - fmin) / (fmax - fmin)
    out_ref[0] = jnp.floor(fn * 255.0) / 255.0


@jax.jit
def kernel(img):
    b, _, h, w = img.shape
    nchunks = h // _ROWS
    flat = img.reshape(b, 3, h * w)

    counts = pl.pallas_call(
        lambda img_ref, cnt_ref: _hist_kernel(w, img_ref, cnt_ref),
        grid=(b, nchunks),
        in_specs=[pl.BlockSpec((1, 3, _ROWS * w), lambda i, c: (i, 0, c))],
        out_specs=pl.BlockSpec((1, _LEVELS, _LEVELS), lambda i, c: (i, 0, 0)),
        out_shape=jax.ShapeDtypeStruct((b, _LEVELS, _LEVELS), jnp.float32),
        compiler_params=pltpu.CompilerParams(
            dimension_semantics=("parallel", "arbitrary")),
    )(flat)

    out = pl.pallas_call(
        _stats_kernel,
        grid=(b,),
        in_specs=[pl.BlockSpec((1, _LEVELS, _LEVELS), lambda i: (i, 0, 0))],
        out_specs=pl.BlockSpec((1, 1, 128), lambda i: (i, 0, 0)),
        out_shape=jax.ShapeDtypeStruct((b, 1, 128), jnp.float32),
        compiler_params=pltpu.CompilerParams(
            dimension_semantics=("parallel",)),
    )(counts)

    feats = out[:, 0, :5]
    return jnp.broadcast_to(feats[:, None, :], (b, 3, 5))
